# baseline (speedup 1.0000x reference)
import functools

import jax
import jax.numpy as jnp
from jax import lax
from jax.experimental import pallas as pl
from jax.experimental.pallas import tpu as pltpu
from jax.experimental.pallas import tpu_sc as plsc

EMB_DIM = 64
NUM_CORES = 2
NUM_SUBCORES = 16
NUM_WORKERS = NUM_CORES * NUM_SUBCORES
CHUNK = 128
NBUF = 5
SELW = 512


def _sel_body(b_ref, p_ref, o_ref):
    blk = b_ref[0]
    lo = blk[:, 0:EMB_DIM]
    hi = blk[:, EMB_DIM:2 * EMB_DIM]
    p = p_ref[0, 0].reshape(SELW, 1) > 0
    o_ref[0] = jnp.transpose(jnp.where(p, hi, lo), (1, 0))


def _tc_select(big, px, num_s, num_b):
    return pl.pallas_call(
        _sel_body,
        grid=(num_s, num_b // SELW),
        in_specs=[
            pl.BlockSpec((1, SELW, 2 * EMB_DIM), lambda s, j: (s, j, 0)),
            pl.BlockSpec((1, 1, SELW), lambda s, j: (s, 0, j)),
        ],
        out_specs=pl.BlockSpec((1, EMB_DIM, SELW), lambda s, j: (s, 0, j)),
        out_shape=jax.ShapeDtypeStruct((num_s, EMB_DIM, num_b), jnp.float32),
    )(big, px)


@functools.lru_cache(maxsize=None)
def _build_gather(num_s, num_b):
    assert num_b == NUM_WORKERS * CHUNK and num_s % NBUF == 0
    mesh = plsc.VectorSubcoreMesh(core_axis_name="c", subcore_axis_name="s")

    @functools.partial(
        pl.kernel,
        out_type=jax.ShapeDtypeStruct((num_s, num_b, 2 * EMB_DIM),
                                      jnp.float32),
        mesh=mesh,
        scratch_types=[
            pltpu.VMEM((num_s, CHUNK), jnp.int32),
            pltpu.VMEM((NBUF, CHUNK), jnp.int32),
            pltpu.VMEM((NBUF, CHUNK, 2 * EMB_DIM), jnp.float32),
            pltpu.SemaphoreType.DMA((NBUF,)),
            pltpu.SemaphoreType.DMA((NBUF,)),
        ],
        compiler_params=pltpu.CompilerParams(use_tc_tiling_on_sc=True),
    )
    def emb_kernel(idx_hbm, table_hbm, out_hbm, idx_v, hidx_v, big_v,
                   gsem, ssem):
        wid = lax.axis_index("s") * NUM_CORES + lax.axis_index("c")
        lane0 = wid * CHUNK

        pltpu.sync_copy(idx_hbm.at[wid], idx_v)

        def fill_hidx(s, b):
            for rg in range(8):
                v = idx_v[s, pl.ds(16 * rg, 16)]
                hidx_v[b, pl.ds(16 * rg, 16)] = lax.shift_right_logical(v, 1)

        def start_gather(b):
            pltpu.async_copy(table_hbm.at[hidx_v.at[b]], big_v.at[b],
                             gsem.at[b])

        def wait_gather(b):
            pltpu.make_async_copy(table_hbm.at[hidx_v.at[b]], big_v.at[b],
                                  gsem.at[b]).wait()

        def start_store(s, b):
            pltpu.async_copy(big_v.at[b],
                             out_hbm.at[s, pl.ds(lane0, CHUNK)],
                             ssem.at[b])

        def wait_store(b):
            pltpu.make_async_copy(big_v.at[b],
                                  out_hbm.at[0, pl.ds(lane0, CHUNK)],
                                  ssem.at[b]).wait()

        for b in range(NBUF):
            fill_hidx(b, b)
            start_gather(b)

        @pl.loop(0, num_s, step=NBUF)
        def _(g):
            for b in range(NBUF):
                s = g + b
                wait_gather(b)
                start_store(s, b)

                @pl.when(s + NBUF < num_s)
                def _():
                    wait_store(b)
                    fill_hidx(s + NBUF, b)
                    start_gather(b)

        for b in range(NBUF):
            wait_store(b)

    return emb_kernel


def kernel(x, table):
    num_b, num_s = x.shape
    xi = x.astype(jnp.int32)
    idx = jnp.transpose(xi, (1, 0)).reshape(num_s, NUM_WORKERS, CHUNK)
    idx = jnp.transpose(idx, (1, 0, 2))
    table2 = table.reshape(table.shape[0] // 2, 2 * EMB_DIM)
    big = _build_gather(num_s, num_b)(idx, table2)
    px = jnp.transpose(xi & 1, (1, 0)).reshape(num_s, 1, num_b)
    out_t = _tc_select(big, px, num_s, num_b)
    return jnp.transpose(out_t, (2, 0, 1))


# speedup vs baseline: 1.0162x; 1.0162x over previous
import functools

import jax
import jax.numpy as jnp
from jax import lax
from jax.experimental import pallas as pl
from jax.experimental.pallas import tpu as pltpu
from jax.experimental.pallas import tpu_sc as plsc

EMB_DIM = 64
NUM_CORES = 2
NUM_SUBCORES = 16
NUM_WORKERS = NUM_CORES * NUM_SUBCORES
CHUNK = 128
NBUF = 2


@functools.lru_cache(maxsize=None)
def _build(num_s, num_b):
    assert num_b == NUM_WORKERS * CHUNK and num_s % NBUF == 0
    mesh = plsc.VectorSubcoreMesh(core_axis_name="c", subcore_axis_name="s")

    @functools.partial(
        pl.kernel,
        out_type=jax.ShapeDtypeStruct((num_s, EMB_DIM, num_b), jnp.float32),
        mesh=mesh,
        scratch_types=[
            pltpu.VMEM((num_s, CHUNK), jnp.int32),
            pltpu.VMEM((NBUF, CHUNK), jnp.int32),
            pltpu.VMEM((NBUF, CHUNK, 1, 2 * EMB_DIM), jnp.float32),
            pltpu.VMEM((NBUF, EMB_DIM, CHUNK), jnp.float32),
            pltpu.SemaphoreType.DMA((NBUF,)),
            pltpu.SemaphoreType.DMA((NBUF,)),
        ],
        compiler_params=pltpu.CompilerParams(needs_layout_passes=False),
    )
    def emb_kernel(idx_hbm, table_hbm, out_hbm, idx_v, hidx_v, big_v,
                   outb_v, gsem, ssem):
        wid = lax.axis_index("s") * NUM_CORES + lax.axis_index("c")
        lane0 = wid * CHUNK

        pltpu.sync_copy(idx_hbm.at[wid], idx_v)

        def fill_hidx(s, b):
            for rg in range(8):
                v = idx_v[s, pl.ds(16 * rg, 16)]
                hidx_v[b, pl.ds(16 * rg, 16)] = lax.shift_right_logical(v, 1)

        def start_gather(b):
            pltpu.async_copy(table_hbm.at[hidx_v.at[b]], big_v.at[b],
                             gsem.at[b])

        def wait_gather(b):
            pltpu.make_async_copy(table_hbm.at[hidx_v.at[b]], big_v.at[b],
                                  gsem.at[b]).wait()

        def start_store(s, b):
            pltpu.async_copy(outb_v.at[b],
                             out_hbm.at[s, :, pl.ds(lane0, CHUNK)],
                             ssem.at[b])

        def wait_store(b):
            pltpu.make_async_copy(outb_v.at[b],
                                  out_hbm.at[0, :, pl.ds(lane0, CHUNK)],
                                  ssem.at[b]).wait()

        def select_transpose(s, b):
            zeros = jnp.full((16,), 0, jnp.int32)
            for jg in range(CHUNK // 16):
                j16 = lax.iota(jnp.int32, 16) + 16 * jg
                c16 = (idx_v[s, pl.ds(16 * jg, 16)] & 1) * EMB_DIM
                for d in range(EMB_DIM):
                    g = plsc.load_gather(big_v.at[b], [j16, zeros, c16 + d])
                    outb_v[b, d, pl.ds(16 * jg, 16)] = g

        for b in range(NBUF):
            fill_hidx(b, b)
            start_gather(b)

        @pl.loop(0, num_s, step=NBUF)
        def _(g):
            for b in range(NBUF):
                s = g + b
                wait_gather(b)

                @pl.when(s >= NBUF)
                def _():
                    wait_store(b)

                select_transpose(s, b)
                start_store(s, b)

                @pl.when(s + NBUF < num_s)
                def _():
                    fill_hidx(s + NBUF, b)
                    start_gather(b)

        for b in range(NBUF):
            wait_store(b)

    return emb_kernel


def kernel(x, table):
    num_b, num_s = x.shape
    xi = x.astype(jnp.int32)
    idx = jnp.transpose(xi, (1, 0)).reshape(num_s, NUM_WORKERS, CHUNK)
    idx = jnp.transpose(idx, (1, 0, 2))
    table3 = table.reshape(table.shape[0] // 2, 1, 2 * EMB_DIM)
    out_t = _build(num_s, num_b)(idx, table3)
    return jnp.transpose(out_t, (2, 0, 1))


# speedup vs baseline: 1.2408x; 1.2209x over previous
import functools

import jax
import jax.numpy as jnp
from jax import lax
from jax.experimental import pallas as pl
from jax.experimental.pallas import tpu as pltpu
from jax.experimental.pallas import tpu_sc as plsc

EMB_DIM = 64
NUM_CORES = 2
NUM_SUBCORES = 16
NUM_WORKERS = NUM_CORES * NUM_SUBCORES
CHUNK = 128
NBUF = 5


@functools.lru_cache(maxsize=None)
def _build(num_idx):
    assert num_idx % (NUM_WORKERS * CHUNK) == 0
    per_worker = num_idx // NUM_WORKERS
    nchunk = per_worker // CHUNK
    assert nchunk % NBUF == 0 and nchunk >= 2 * NBUF

    mesh = plsc.VectorSubcoreMesh(core_axis_name="c", subcore_axis_name="s")

    @functools.partial(
        pl.kernel,
        out_type=jax.ShapeDtypeStruct((num_idx, EMB_DIM), jnp.float32),
        mesh=mesh,
        scratch_types=[
            pltpu.VMEM((nchunk, CHUNK), jnp.int32),
            pltpu.VMEM((NBUF, CHUNK, EMB_DIM), jnp.float32),
            pltpu.SemaphoreType.DMA((NBUF,)),
            pltpu.SemaphoreType.DMA((NBUF,)),
        ],
        compiler_params=pltpu.CompilerParams(use_tc_tiling_on_sc=False),
    )
    def emb_kernel(idx_hbm, table_hbm, out_hbm, idx_v, rows_v, gsem, ssem):
        wid = lax.axis_index("s") * NUM_CORES + lax.axis_index("c")
        row_base = wid * per_worker

        pltpu.sync_copy(idx_hbm.at[wid], idx_v)

        def start_gather(j, b):
            pltpu.async_copy(table_hbm.at[idx_v.at[j]], rows_v.at[b],
                             gsem.at[b])

        def wait_gather(b):
            pltpu.make_async_copy(table_hbm.at[idx_v.at[0]], rows_v.at[b],
                                  gsem.at[b]).wait()

        def start_store(j, b):
            pltpu.async_copy(rows_v.at[b],
                             out_hbm.at[pl.ds(row_base + j * CHUNK, CHUNK)],
                             ssem.at[b])

        def wait_store(b):
            pltpu.make_async_copy(
                rows_v.at[b],
                out_hbm.at[pl.ds(row_base, CHUNK)],
                ssem.at[b]).wait()

        for b in range(NBUF):
            start_gather(b, b)

        @pl.loop(0, nchunk - NBUF, step=NBUF)
        def _(g):
            for b in range(NBUF):
                j = g + b
                wait_gather(b)
                start_store(j, b)
                wait_store(b)
                start_gather(j + NBUF, b)

        for b in range(NBUF):
            wait_gather(b)
            start_store(nchunk - NBUF + b, b)
        for b in range(NBUF):
            wait_store(b)

    return emb_kernel


def kernel(x, table):
    num_idx = x.size
    idx = x.reshape(NUM_WORKERS, num_idx // (NUM_WORKERS * CHUNK), CHUNK)
    idx = idx.astype(jnp.int32)
    out = _build(num_idx)(idx, table)
    return out.reshape(x.shape + (EMB_DIM,))
